# Initial kernel scaffold; baseline (speedup 1.0000x reference)
import jax
import jax.numpy as jnp
from jax.experimental import pallas as pl


def kernel(x, W1, b1, W2, b2, codebook, W3, b3, W4, b4):
    raise NotImplementedError("write your pallas kernel here")


# speedup vs baseline: 1.0039x; 1.0039x over previous
import functools

import jax
import jax.numpy as jnp
from jax.experimental import pallas as pl

B = 4096
T = 16
D = 3
N_CODES = 1024
CODE_DIM = 32
N_TOKENS = 8
HIDDEN = 256

BB = 256


def _vqvae_kernel(x_ref, w1_ref, b1_ref, w2_ref, b2_ref, cb_ref, cbt_ref,
                  cbsq_ref, w3_ref, b3_ref, w4_ref, b4_ref,
                  recon_ref, zq_ref, idx_ref):
    x = x_ref[...]
    h = jnp.maximum(
        jnp.dot(x, w1_ref[...], preferred_element_type=jnp.float32)
        + b1_ref[...], 0.0)
    z = jnp.dot(h, w2_ref[...], preferred_element_type=jnp.float32) + b2_ref[...]

    cbt = cbt_ref[...]
    cbsq = cbsq_ref[...]
    cb = cb_ref[...]

    zq_cols = []
    for t in range(N_TOKENS):
        zt = z[:, t * CODE_DIM:(t + 1) * CODE_DIM]
        mu = jnp.mean(zt, axis=1, keepdims=True)
        zc = zt - mu
        var = jnp.mean(zc * zc, axis=1, keepdims=True)
        ze = zc * jax.lax.rsqrt(var + 1e-5)
        d = cbsq - 2.0 * jnp.dot(ze, cbt, preferred_element_type=jnp.float32)
        idx = jnp.argmin(d, axis=1).astype(jnp.int32)
        idx_ref[:, t] = idx
        onehot = (jax.lax.broadcasted_iota(jnp.int32, (BB, N_CODES), 1)
                  == idx[:, None]).astype(jnp.float32)
        zq_raw = jnp.dot(onehot, cb, preferred_element_type=jnp.float32)
        zq_cols.append(ze + (zq_raw - ze))
    zq = jnp.concatenate(zq_cols, axis=1)
    zq_ref[...] = zq

    h2 = jnp.maximum(
        jnp.dot(zq, w3_ref[...], preferred_element_type=jnp.float32)
        + b3_ref[...], 0.0)
    recon_ref[...] = (
        jnp.dot(h2, w4_ref[...], preferred_element_type=jnp.float32)
        + b4_ref[...])


@functools.partial(jax.jit, static_argnames=())
def _run(x2, W1, b1, W2, b2, codebook, W3, b3, W4, b4):
    cbt = codebook.T
    cbsq = jnp.sum(codebook * codebook, axis=1)[None, :]
    grid = (B // BB,)

    def bspec(shape):
        return pl.BlockSpec(shape, lambda i: (0,) * len(shape))

    recon, zq, idx = pl.pallas_call(
        _vqvae_kernel,
        grid=grid,
        in_specs=[
            pl.BlockSpec((BB, T * D), lambda i: (i, 0)),
            bspec((T * D, HIDDEN)),
            bspec((1, HIDDEN)),
            bspec((HIDDEN, N_TOKENS * CODE_DIM)),
            bspec((1, N_TOKENS * CODE_DIM)),
            bspec((N_CODES, CODE_DIM)),
            bspec((CODE_DIM, N_CODES)),
            bspec((1, N_CODES)),
            bspec((N_TOKENS * CODE_DIM, HIDDEN)),
            bspec((1, HIDDEN)),
            bspec((HIDDEN, T * D)),
            bspec((1, T * D)),
        ],
        out_specs=[
            pl.BlockSpec((BB, T * D), lambda i: (i, 0)),
            pl.BlockSpec((BB, N_TOKENS * CODE_DIM), lambda i: (i, 0)),
            pl.BlockSpec((BB, N_TOKENS), lambda i: (i, 0)),
        ],
        out_shape=[
            jax.ShapeDtypeStruct((B, T * D), jnp.float32),
            jax.ShapeDtypeStruct((B, N_TOKENS * CODE_DIM), jnp.float32),
            jax.ShapeDtypeStruct((B, N_TOKENS), jnp.int32),
        ],
    )(x2, W1, b1[None, :], W2, b2[None, :], codebook, cbt, cbsq,
      W3, b3[None, :], W4, b4[None, :])
    return recon, zq, idx


def kernel(x, W1, b1, W2, b2, codebook, W3, b3, W4, b4):
    x2 = x.reshape(B, T * D)
    recon, zq, idx = _run(x2, W1, b1, W2, b2, codebook, W3, b3, W4, b4)
    return (recon.reshape(B, T, D),
            zq.reshape(B, N_TOKENS, CODE_DIM),
            idx)


# speedup vs baseline: 2.2167x; 2.2080x over previous
import functools

import jax
import jax.numpy as jnp
import numpy as np
from jax.experimental import pallas as pl
from jax.experimental.pallas import tpu as pltpu

B = 4096
T = 16
D = 3
N_CODES = 1024
CODE_DIM = 32
N_TOKENS = 8
HIDDEN = 256

BB = 256


def _vqvae_kernel(x_ref, w1_ref, b1_ref, w2_ref, b2_ref, cb_ref, cbt_ref,
                  cbsq_ref, mavg_ref, w3_ref, b3_ref, w4_ref, b4_ref,
                  recon_ref, zq_ref, idx_ref):
    x = x_ref[...]
    h = jnp.maximum(
        jnp.dot(x, w1_ref[...], preferred_element_type=jnp.float32)
        + b1_ref[...], 0.0)
    z = jnp.dot(h, w2_ref[...], preferred_element_type=jnp.float32) + b2_ref[...]

    mavg = mavg_ref[...]
    mu = jnp.dot(z, mavg, preferred_element_type=jnp.float32)
    zc = z - mu
    var = jnp.dot(zc * zc, mavg, preferred_element_type=jnp.float32)
    ze = zc * jax.lax.rsqrt(var + 1e-5)

    cbt = cbt_ref[...]
    cbsq = cbsq_ref[...]
    cb = cb_ref[...]

    ds = []
    for t in range(N_TOKENS):
        zet = ze[:, t * CODE_DIM:(t + 1) * CODE_DIM]
        ds.append(cbsq - 2.0 * jnp.dot(zet, cbt,
                                       preferred_element_type=jnp.float32))

    lanes = jax.lax.broadcasted_iota(jnp.int32, (BB, N_CODES), 1)
    idxs = []
    for t in range(N_TOKENS):
        d = ds[t]
        m = jnp.min(d, axis=1, keepdims=True)
        idx = jnp.min(jnp.where(d <= m, lanes, N_CODES), axis=1)
        idxs.append(idx)
        idx_ref[:, t] = idx

    zq_cols = []
    for t in range(N_TOKENS):
        onehot = (lanes == idxs[t][:, None]).astype(jnp.float32)
        zq_raw = jnp.dot(onehot, cb, preferred_element_type=jnp.float32)
        zet = ze[:, t * CODE_DIM:(t + 1) * CODE_DIM]
        zq_cols.append(zet + (zq_raw - zet))
    zq = jnp.concatenate(zq_cols, axis=1)
    zq_ref[...] = zq

    h2 = jnp.maximum(
        jnp.dot(zq, w3_ref[...], preferred_element_type=jnp.float32)
        + b3_ref[...], 0.0)
    recon_ref[...] = (
        jnp.dot(h2, w4_ref[...], preferred_element_type=jnp.float32)
        + b4_ref[...])


@jax.jit
def _run(x2, W1, b1, W2, b2, codebook, W3, b3, W4, b4):
    cbt = codebook.T
    cbsq = jnp.sum(codebook * codebook, axis=1)[None, :]
    mavg = jnp.asarray(
        np.kron(np.eye(N_TOKENS, dtype=np.float32),
                np.full((CODE_DIM, CODE_DIM), 1.0 / CODE_DIM,
                        dtype=np.float32)))
    grid = (B // BB,)

    def bspec(shape):
        return pl.BlockSpec(shape, lambda i: (0,) * len(shape))

    recon, zq, idx = pl.pallas_call(
        _vqvae_kernel,
        grid=grid,
        in_specs=[
            pl.BlockSpec((BB, T * D), lambda i: (i, 0)),
            bspec((T * D, HIDDEN)),
            bspec((1, HIDDEN)),
            bspec((HIDDEN, N_TOKENS * CODE_DIM)),
            bspec((1, N_TOKENS * CODE_DIM)),
            bspec((N_CODES, CODE_DIM)),
            bspec((CODE_DIM, N_CODES)),
            bspec((1, N_CODES)),
            bspec((N_TOKENS * CODE_DIM, N_TOKENS * CODE_DIM)),
            bspec((N_TOKENS * CODE_DIM, HIDDEN)),
            bspec((1, HIDDEN)),
            bspec((HIDDEN, T * D)),
            bspec((1, T * D)),
        ],
        out_specs=[
            pl.BlockSpec((BB, T * D), lambda i: (i, 0)),
            pl.BlockSpec((BB, N_TOKENS * CODE_DIM), lambda i: (i, 0)),
            pl.BlockSpec((BB, N_TOKENS), lambda i: (i, 0)),
        ],
        out_shape=[
            jax.ShapeDtypeStruct((B, T * D), jnp.float32),
            jax.ShapeDtypeStruct((B, N_TOKENS * CODE_DIM), jnp.float32),
            jax.ShapeDtypeStruct((B, N_TOKENS), jnp.int32),
        ],
        compiler_params=pltpu.CompilerParams(
            dimension_semantics=("parallel",)),
    )(x2, W1, b1[None, :], W2, b2[None, :], codebook, cbt, cbsq, mavg,
      W3, b3[None, :], W4, b4[None, :])
    return recon, zq, idx


def kernel(x, W1, b1, W2, b2, codebook, W3, b3, W4, b4):
    x2 = x.reshape(B, T * D)
    recon, zq, idx = _run(x2, W1, b1, W2, b2, codebook, W3, b3, W4, b4)
    return (recon.reshape(B, T, D),
            zq.reshape(B, N_TOKENS, CODE_DIM),
            idx)


# speedup vs baseline: 2.2243x; 1.0035x over previous
import functools

import jax
import jax.numpy as jnp
import numpy as np
from jax.experimental import pallas as pl
from jax.experimental.pallas import tpu as pltpu

B = 4096
T = 16
D = 3
N_CODES = 1024
CODE_DIM = 32
N_TOKENS = 8
HIDDEN = 256

BB = 256


def _vqvae_kernel(x_ref, w1_ref, b1_ref, w2_ref, b2_ref, cb_ref, cbt_ref,
                  cbsq_ref, mavg_ref, w3_ref, b3_ref, w4_ref, b4_ref,
                  recon_ref, zq_ref, idx_ref):
    x = x_ref[...]
    h = jnp.maximum(
        jnp.dot(x, w1_ref[...], preferred_element_type=jnp.float32)
        + b1_ref[...], 0.0)
    z = jnp.dot(h, w2_ref[...], preferred_element_type=jnp.float32) + b2_ref[...]

    mavg = mavg_ref[...]
    mu = jnp.dot(z, mavg, preferred_element_type=jnp.float32,
                 precision=jax.lax.Precision.HIGHEST)
    zc = z - mu
    var = jnp.dot(zc * zc, mavg, preferred_element_type=jnp.float32,
                  precision=jax.lax.Precision.HIGHEST)
    ze = zc * jax.lax.rsqrt(var + 1e-5)

    cbt = cbt_ref[...]
    cbsq = cbsq_ref[...]
    cb = cb_ref[...]

    ds = []
    for t in range(N_TOKENS):
        zet = ze[:, t * CODE_DIM:(t + 1) * CODE_DIM]
        ds.append(cbsq - 2.0 * jnp.dot(zet, cbt,
                                       preferred_element_type=jnp.float32))

    lanes = jax.lax.broadcasted_iota(jnp.int32, (BB, N_CODES), 1)
    idxs = []
    for t in range(N_TOKENS):
        d = ds[t]
        idx = jnp.argmin(d, axis=1).astype(jnp.int32)
        idxs.append(idx)
        idx_ref[:, t] = idx

    zq_cols = []
    for t in range(N_TOKENS):
        onehot = (lanes == idxs[t][:, None]).astype(jnp.float32)
        zq_raw = jnp.dot(onehot, cb, preferred_element_type=jnp.float32)
        zet = ze[:, t * CODE_DIM:(t + 1) * CODE_DIM]
        zq_cols.append(zet + (zq_raw - zet))
    zq = jnp.concatenate(zq_cols, axis=1)
    zq_ref[...] = zq

    h2 = jnp.maximum(
        jnp.dot(zq, w3_ref[...], preferred_element_type=jnp.float32)
        + b3_ref[...], 0.0)
    recon_ref[...] = (
        jnp.dot(h2, w4_ref[...], preferred_element_type=jnp.float32)
        + b4_ref[...])


@jax.jit
def _run(x2, W1, b1, W2, b2, codebook, W3, b3, W4, b4):
    cbt = codebook.T
    cbsq = jnp.sum(codebook * codebook, axis=1)[None, :]
    mavg = jnp.asarray(
        np.kron(np.eye(N_TOKENS, dtype=np.float32),
                np.full((CODE_DIM, CODE_DIM), 1.0 / CODE_DIM,
                        dtype=np.float32)))
    grid = (B // BB,)

    def bspec(shape):
        return pl.BlockSpec(shape, lambda i: (0,) * len(shape))

    recon, zq, idx = pl.pallas_call(
        _vqvae_kernel,
        grid=grid,
        in_specs=[
            pl.BlockSpec((BB, T * D), lambda i: (i, 0)),
            bspec((T * D, HIDDEN)),
            bspec((1, HIDDEN)),
            bspec((HIDDEN, N_TOKENS * CODE_DIM)),
            bspec((1, N_TOKENS * CODE_DIM)),
            bspec((N_CODES, CODE_DIM)),
            bspec((CODE_DIM, N_CODES)),
            bspec((1, N_CODES)),
            bspec((N_TOKENS * CODE_DIM, N_TOKENS * CODE_DIM)),
            bspec((N_TOKENS * CODE_DIM, HIDDEN)),
            bspec((1, HIDDEN)),
            bspec((HIDDEN, T * D)),
            bspec((1, T * D)),
        ],
        out_specs=[
            pl.BlockSpec((BB, T * D), lambda i: (i, 0)),
            pl.BlockSpec((BB, N_TOKENS * CODE_DIM), lambda i: (i, 0)),
            pl.BlockSpec((BB, N_TOKENS), lambda i: (i, 0)),
        ],
        out_shape=[
            jax.ShapeDtypeStruct((B, T * D), jnp.float32),
            jax.ShapeDtypeStruct((B, N_TOKENS * CODE_DIM), jnp.float32),
            jax.ShapeDtypeStruct((B, N_TOKENS), jnp.int32),
        ],
        compiler_params=pltpu.CompilerParams(
            dimension_semantics=("arbitrary",)),
    )(x2, W1, b1[None, :], W2, b2[None, :], codebook, cbt, cbsq, mavg,
      W3, b3[None, :], W4, b4[None, :])
    return recon, zq, idx


def kernel(x, W1, b1, W2, b2, codebook, W3, b3, W4, b4):
    x2 = x.reshape(B, T * D)
    recon, zq, idx = _run(x2, W1, b1, W2, b2, codebook, W3, b3, W4, b4)
    return (recon.reshape(B, T, D),
            zq.reshape(B, N_TOKENS, CODE_DIM),
            idx)


# speedup vs baseline: 2.3693x; 1.0652x over previous
import jax
import jax.numpy as jnp
import numpy as np
from jax.experimental import pallas as pl
from jax.experimental.pallas import tpu as pltpu

B = 4096
T = 16
D = 3
N_CODES = 1024
CODE_DIM = 32
N_TOKENS = 8
HIDDEN = 256

BB = 256


def _vqvae_kernel(x_ref, w1_ref, b1_ref, w2_ref, b2_ref, cbt_ref,
                  cbsq_ref, gmat_ref, mavg_ref, w3_ref, b3_ref, w4_ref,
                  b4_ref, recon_ref, zq_ref, idx_ref):
    x = x_ref[...]
    h = jnp.maximum(
        jnp.dot(x, w1_ref[...], preferred_element_type=jnp.float32)
        + b1_ref[...], 0.0)
    z = jnp.dot(h, w2_ref[...], preferred_element_type=jnp.float32) + b2_ref[...]

    mavg = mavg_ref[...]
    mu = jnp.dot(z, mavg, preferred_element_type=jnp.float32,
                 precision=jax.lax.Precision.HIGHEST)
    zc = z - mu
    var = jnp.dot(zc * zc, mavg, preferred_element_type=jnp.float32,
                  precision=jax.lax.Precision.HIGHEST)
    ze = zc * jax.lax.rsqrt(var + 1e-5)

    cbt = cbt_ref[...]
    cbsq = cbsq_ref[...]
    gmat = gmat_ref[...]

    ds = []
    for t in range(N_TOKENS):
        zet = ze[:, t * CODE_DIM:(t + 1) * CODE_DIM]
        ds.append(cbsq - 2.0 * jnp.dot(zet, cbt,
                                       preferred_element_type=jnp.float32))

    ms = [jnp.min(d, axis=1, keepdims=True) for d in ds]

    gs = []
    for t in range(N_TOKENS):
        onehot = (ds[t] <= ms[t]).astype(jnp.float32)
        gs.append(jnp.dot(onehot, gmat, preferred_element_type=jnp.float32))

    zq_cols = []
    idx_cols = []
    for t in range(N_TOKENS):
        zq_raw = gs[t][:, :CODE_DIM]
        idx_cols.append(gs[t][:, CODE_DIM:CODE_DIM + 1])
        zet = ze[:, t * CODE_DIM:(t + 1) * CODE_DIM]
        zq_cols.append(zet + (zq_raw - zet))
    zq = jnp.concatenate(zq_cols, axis=1)
    zq_ref[...] = zq
    idx_ref[...] = (jnp.concatenate(idx_cols, axis=1)
                    + 0.5).astype(jnp.int32)

    h2 = jnp.maximum(
        jnp.dot(zq, w3_ref[...], preferred_element_type=jnp.float32)
        + b3_ref[...], 0.0)
    recon_ref[...] = (
        jnp.dot(h2, w4_ref[...], preferred_element_type=jnp.float32)
        + b4_ref[...])


@jax.jit
def _run(x2, W1, b1, W2, b2, codebook, W3, b3, W4, b4):
    cbt = codebook.T
    cbsq = jnp.sum(codebook * codebook, axis=1)[None, :]
    iota_col = jnp.arange(N_CODES, dtype=jnp.float32)[:, None]
    gmat = jnp.concatenate(
        [codebook, jnp.tile(iota_col, (1, 8))], axis=1)
    mavg = jnp.asarray(
        np.kron(np.eye(N_TOKENS, dtype=np.float32),
                np.full((CODE_DIM, CODE_DIM), 1.0 / CODE_DIM,
                        dtype=np.float32)))
    grid = (B // BB,)

    def bspec(shape):
        return pl.BlockSpec(shape, lambda i: (0,) * len(shape))

    recon, zq, idx = pl.pallas_call(
        _vqvae_kernel,
        grid=grid,
        in_specs=[
            pl.BlockSpec((BB, T * D), lambda i: (i, 0)),
            bspec((T * D, HIDDEN)),
            bspec((1, HIDDEN)),
            bspec((HIDDEN, N_TOKENS * CODE_DIM)),
            bspec((1, N_TOKENS * CODE_DIM)),
            bspec((CODE_DIM, N_CODES)),
            bspec((1, N_CODES)),
            bspec((N_CODES, CODE_DIM + 8)),
            bspec((N_TOKENS * CODE_DIM, N_TOKENS * CODE_DIM)),
            bspec((N_TOKENS * CODE_DIM, HIDDEN)),
            bspec((1, HIDDEN)),
            bspec((HIDDEN, T * D)),
            bspec((1, T * D)),
        ],
        out_specs=[
            pl.BlockSpec((BB, T * D), lambda i: (i, 0)),
            pl.BlockSpec((BB, N_TOKENS * CODE_DIM), lambda i: (i, 0)),
            pl.BlockSpec((BB, N_TOKENS), lambda i: (i, 0)),
        ],
        out_shape=[
            jax.ShapeDtypeStruct((B, T * D), jnp.float32),
            jax.ShapeDtypeStruct((B, N_TOKENS * CODE_DIM), jnp.float32),
            jax.ShapeDtypeStruct((B, N_TOKENS), jnp.int32),
        ],
        compiler_params=pltpu.CompilerParams(
            dimension_semantics=("arbitrary",)),
    )(x2, W1, b1[None, :], W2, b2[None, :], cbt, cbsq, gmat, mavg,
      W3, b3[None, :], W4, b4[None, :])
    return recon, zq, idx


def kernel(x, W1, b1, W2, b2, codebook, W3, b3, W4, b4):
    x2 = x.reshape(B, T * D)
    recon, zq, idx = _run(x2, W1, b1, W2, b2, codebook, W3, b3, W4, b4)
    return (recon.reshape(B, T, D),
            zq.reshape(B, N_TOKENS, CODE_DIM),
            idx)
